# Initial kernel scaffold; baseline (speedup 1.0000x reference)
import jax
import jax.numpy as jnp
from jax.experimental import pallas as pl


def kernel(pp0, pp1, pp2, pf0, pf1, pf2, gp0, gp1, gp2, gf0, gf1, gf2, params):
    raise NotImplementedError("write your pallas kernel here")


# speedup vs baseline: 16.4227x; 16.4227x over previous
import functools

import jax
import jax.numpy as jnp
from jax import lax
from jax.experimental import pallas as pl
from jax.experimental.pallas import tpu as pltpu
from jax.experimental.pallas import tpu_sc as plsc

_F32 = jnp.float32
_KNN_TN = 128
_MLP_TN = 128


def _proj_body(fq_ref, fs_ref, pq_ref, ps_ref, wq_ref, wk_ref, wv_ref,
               bq_ref, bk_ref, bv_ref, eye_ref,
               q_ref, ke_ref, v_ref, pqr_ref, psr_ref):
    cdims = (((0,), (1,)), ((), ()))
    fq = fq_ref[0]
    fs = fs_ref[0]
    q_ref[0] = lax.dot_general(fq, wq_ref[...], cdims,
                               preferred_element_type=_F32) + bq_ref[...]
    ke_ref[0] = lax.dot_general(fs, wk_ref[...], cdims,
                                preferred_element_type=_F32) + bk_ref[...]
    v_ref[0] = lax.dot_general(fs, wv_ref[...], cdims,
                               preferred_element_type=_F32) + bv_ref[...]
    pqr_ref[0] = lax.dot_general(pq_ref[0], eye_ref[...], cdims,
                                 preferred_element_type=_F32)
    psr_ref[0] = lax.dot_general(ps_ref[0], eye_ref[...], cdims,
                                 preferred_element_type=_F32)


def _proj(fq, fs, pq, ps, wq, wk, wv, bq, bk, bv):
    b, c, n = fq.shape
    m = fs.shape[2]
    eye = jnp.eye(16, 3, dtype=_F32)
    full = lambda s: pl.BlockSpec(s, lambda i: (0,) * len(s))
    return pl.pallas_call(
        _proj_body,
        grid=(b,),
        in_specs=[
            pl.BlockSpec((1, c, n), lambda i: (i, 0, 0)),
            pl.BlockSpec((1, c, m), lambda i: (i, 0, 0)),
            pl.BlockSpec((1, 3, n), lambda i: (i, 0, 0)),
            pl.BlockSpec((1, 3, m), lambda i: (i, 0, 0)),
            full((64, c)), full((64, c)), full((64, c)),
            full((1, 64)), full((1, 64)), full((1, 64)),
            full((16, 3)),
        ],
        out_specs=[
            pl.BlockSpec((1, n, 64), lambda i: (i, 0, 0)),
            pl.BlockSpec((1, m, 64), lambda i: (i, 0, 0)),
            pl.BlockSpec((1, m, 64), lambda i: (i, 0, 0)),
            pl.BlockSpec((1, n, 16), lambda i: (i, 0, 0)),
            pl.BlockSpec((1, m, 16), lambda i: (i, 0, 0)),
        ],
        out_shape=[
            jax.ShapeDtypeStruct((b, n, 64), _F32),
            jax.ShapeDtypeStruct((b, m, 64), _F32),
            jax.ShapeDtypeStruct((b, m, 64), _F32),
            jax.ShapeDtypeStruct((b, n, 16), _F32),
            jax.ShapeDtypeStruct((b, m, 16), _F32),
        ],
    )(fq, fs, pq, ps, wq, wk, wv, bq[None], bk[None], bv[None], eye)


def _knn_body(k, m, pq_ref, ps_ref, idx_ref):
    tn = pq_ref.shape[2]
    pq = pq_ref[0]
    ps = ps_ref[0]
    cross = lax.dot_general(pq, ps, (((0,), (0,)), ((), ())),
                            preferred_element_type=_F32)
    sn = jnp.sum(ps * ps, axis=0, keepdims=True)
    d = sn - 2.0 * cross
    iota = lax.broadcasted_iota(jnp.int32, (tn, m), 1)
    cols = []
    for j in range(k):
        dmin = jnp.min(d, axis=1, keepdims=True)
        sel = jnp.where(d == dmin, iota, m)
        ik = jnp.min(sel, axis=1, keepdims=True)
        cols.append(ik)
        if j + 1 < k:
            d = jnp.where(iota == ik, jnp.float32(jnp.inf), d)
    bi = pl.program_id(0)
    idx_ref[0] = jnp.concatenate(cols, axis=1) + bi * m


def _knn(k, pq, ps):
    b, _, n = pq.shape
    m = ps.shape[2]
    tn = min(_KNN_TN, n)
    return pl.pallas_call(
        functools.partial(_knn_body, k, m),
        grid=(b, n // tn),
        in_specs=[
            pl.BlockSpec((1, 3, tn), lambda i, j: (i, 0, j)),
            pl.BlockSpec((1, 3, m), lambda i, j: (i, 0, 0)),
        ],
        out_specs=pl.BlockSpec((1, tn, k), lambda i, j: (i, j, 0)),
        out_shape=jax.ShapeDtypeStruct((b, n, k), jnp.int32),
    )(pq, ps)


def _sc_gather(ke_t, v_t, ps_t, idx_flat):
    r = idx_flat.shape[0]
    nw = 32
    ch = 128
    n_per_w = r // nw
    n_it = n_per_w // ch
    mesh = plsc.VectorSubcoreMesh(core_axis_name="c", subcore_axis_name="s")

    @functools.partial(
        pl.kernel, mesh=mesh,
        compiler_params=pltpu.CompilerParams(use_tc_tiling_on_sc=False),
        out_type=(jax.ShapeDtypeStruct((r, 64), _F32),
                  jax.ShapeDtypeStruct((r, 64), _F32),
                  jax.ShapeDtypeStruct((r, 16), _F32)),
        scratch_types=[
            pltpu.VMEM((ch,), jnp.int32),
            pltpu.VMEM((ch, 64), _F32),
            pltpu.VMEM((ch, 64), _F32),
            pltpu.VMEM((ch, 16), _F32),
            pltpu.SemaphoreType.DMA,
            pltpu.SemaphoreType.DMA,
            pltpu.SemaphoreType.DMA,
        ],
    )
    def gk(ke_hbm, v_hbm, ps_hbm, idx_hbm, oke, ov, ops,
           idx_v, ke_v, v_v, ps_v, s1, s2, s3):
        wid = lax.axis_index("s") * 2 + lax.axis_index("c")
        base = wid * n_per_w

        def body(i, carry):
            off = pl.multiple_of(base + i * ch, ch)
            pltpu.sync_copy(idx_hbm.at[pl.ds(off, ch)], idx_v)
            c1 = pltpu.async_copy(ke_hbm.at[idx_v], ke_v, s1)
            c2 = pltpu.async_copy(v_hbm.at[idx_v], v_v, s2)
            c3 = pltpu.async_copy(ps_hbm.at[idx_v], ps_v, s3)
            c1.wait()
            c2.wait()
            c3.wait()
            pltpu.sync_copy(ke_v, oke.at[pl.ds(off, ch)])
            pltpu.sync_copy(v_v, ov.at[pl.ds(off, ch)])
            pltpu.sync_copy(ps_v, ops.at[pl.ds(off, ch)])
            return carry

        lax.fori_loop(0, n_it, body, 0)

    return gk(ke_t, v_t, ps_t, idx_flat)


def _mlp_body(k, q_ref, pqr_ref, kg_ref, vg_ref, psg_ref, fq_ref,
              pw1_ref, s1_ref, t1_ref, pw2_ref, pb2_ref,
              aw1_ref, s2_ref, t2_ref, aw2_ref, ab2_ref,
              ew_ref, eb_ref, out_ref):
    tn = q_ref.shape[1]
    ktn = k * tn
    cT = (((1,), (1,)), ((), ()))
    kg = kg_ref[0].reshape(ktn, 64)
    psg = psg_ref[0].reshape(ktn, 16)
    qrep = jnp.broadcast_to(q_ref[0][None], (k, tn, 64)).reshape(ktn, 64)
    pqrep = jnp.broadcast_to(pqr_ref[0][None], (k, tn, 16)).reshape(ktn, 16)

    psrel = pqrep - psg
    pe = lax.dot_general(psrel, pw1_ref[...], cT,
                         preferred_element_type=_F32)
    pe = jnp.maximum(pe * s1_ref[...] + t1_ref[...], 0.0)
    pe = lax.dot_general(pe, pw2_ref[...], cT,
                         preferred_element_type=_F32) + pb2_ref[...]

    ain = qrep - kg + pe
    h = lax.dot_general(ain, aw1_ref[...], cT, preferred_element_type=_F32)
    h = jnp.maximum(h * s2_ref[...] + t2_ref[...], 0.0)
    a = lax.dot_general(h, aw2_ref[...], cT,
                        preferred_element_type=_F32) + ab2_ref[...]

    a3 = a.reshape(k, tn, 64)
    mx = jnp.max(a3, axis=0)
    e3 = jnp.exp(a3 - mx[None])
    w3 = e3 / jnp.sum(e3, axis=0)[None]
    vpe = vg_ref[0] + pe.reshape(k, tn, 64)
    agg = jnp.sum(w3 * vpe, axis=0)

    outc = lax.dot_general(ew_ref[...], agg, cT,
                           preferred_element_type=_F32)
    out_ref[0] = outc + eb_ref[...] + fq_ref[0]


def _mlp(k, q, pqr, gke, gv, gps, fq, pw1, s1, t1, pw2, pb2,
         aw1, s2, t2, aw2, ab2, ew, eb):
    b, c, n = fq.shape
    tn = min(_MLP_TN, n)
    full = lambda s: pl.BlockSpec(s, lambda i, j: (0,) * len(s))
    return pl.pallas_call(
        functools.partial(_mlp_body, k),
        grid=(b, n // tn),
        in_specs=[
            pl.BlockSpec((1, tn, 64), lambda i, j: (i, j, 0)),
            pl.BlockSpec((1, tn, 16), lambda i, j: (i, j, 0)),
            pl.BlockSpec((1, k, tn, 64), lambda i, j: (i, 0, j, 0)),
            pl.BlockSpec((1, k, tn, 64), lambda i, j: (i, 0, j, 0)),
            pl.BlockSpec((1, k, tn, 16), lambda i, j: (i, 0, j, 0)),
            pl.BlockSpec((1, c, tn), lambda i, j: (i, 0, j)),
            full((64, 16)), full((1, 64)), full((1, 64)),
            full((64, 64)), full((1, 64)),
            full((256, 64)), full((1, 256)), full((1, 256)),
            full((64, 256)), full((1, 64)),
            full((c, 64)), full((c, 1)),
        ],
        out_specs=pl.BlockSpec((1, c, tn), lambda i, j: (i, 0, j)),
        out_shape=jax.ShapeDtypeStruct((b, c, n), _F32),
    )(q, pqr, gke, gv, gps, fq,
      pw1, s1, t1, pw2, pb2, aw1, s2, t2, aw2, ab2, ew, eb)


def _fold_bn(g, be, m, v, bias):
    scale = g * lax.rsqrt(v + 1e-5)
    shift = be - m * scale + bias * scale
    return scale[None], shift[None]


def _vfa_scale(p, k, pq, fq, ps, fs):
    b, c, n = fq.shape
    m = fs.shape[2]

    q, ke, v, pqr, psr = _proj(fq, fs, pq, ps,
                               p['wq'], p['wk'], p['wv'],
                               p['bq'], p['bk'], p['bv'])
    idx = _knn(k, pq, ps)
    idx_flat = jnp.transpose(idx, (0, 2, 1)).reshape(b * k * n)
    gke, gv, gps = _sc_gather(ke.reshape(b * m, 64),
                              v.reshape(b * m, 64),
                              psr.reshape(b * m, 16), idx_flat)
    gke = gke.reshape(b, k, n, 64)
    gv = gv.reshape(b, k, n, 64)
    gps = gps.reshape(b, k, n, 16)

    pw1 = jnp.pad(p['pw1'], ((0, 0), (0, 13)))
    s1, t1 = _fold_bn(p['pg'], p['pbe'], p['pm'], p['pv'], p['pb1'])
    s2, t2 = _fold_bn(p['ag'], p['abe'], p['am'], p['av'], p['ab1'])
    return _mlp(k, q, pqr, gke, gv, gps, fq,
                pw1, s1, t1, p['pw2'], p['pb2'][None],
                p['aw1'], s2, t2, p['aw2'], p['ab2'][None],
                p['ew'], p['eb'][:, None])


def kernel(pp0, pp1, pp2, pf0, pf1, pf2, gp0, gp1, gp2, gf0, gf1, gf2, params):
    pp = [pp0, pp1, pp2]
    pf = [pf0, pf1, pf2]
    gp = [gp0, gp1, gp2]
    gf = [gf0, gf1, gf2]
    knns = [16, 12, 8]
    pre_pos = []
    pre_f = []
    for i in range(2, -1, -1):
        f = _vfa_scale(params[i], knns[i], pp[i], pf[i], gp[i], gf[i])
        pre_pos.append(pp[i])
        pre_f.append(f)
    return (tuple(pre_pos), tuple(pre_f))


# speedup vs baseline: 19.3182x; 1.1763x over previous
import functools

import jax
import jax.numpy as jnp
from jax import lax
from jax.experimental import pallas as pl
from jax.experimental.pallas import tpu as pltpu
from jax.experimental.pallas import tpu_sc as plsc

_F32 = jnp.float32
_KNN_TN = 128
_MLP_TN = 128


def _proj_body(fq_ref, fs_ref, pq_ref, ps_ref, wq_ref, wk_ref, wv_ref,
               bq_ref, bk_ref, bv_ref, eye_ref,
               q_ref, ke_ref, v_ref, pqr_ref, psr_ref):
    cdims = (((0,), (1,)), ((), ()))
    fq = fq_ref[0]
    fs = fs_ref[0]
    q_ref[0] = lax.dot_general(fq, wq_ref[...], cdims,
                               preferred_element_type=_F32) + bq_ref[...]
    ke_ref[0] = lax.dot_general(fs, wk_ref[...], cdims,
                                preferred_element_type=_F32) + bk_ref[...]
    v_ref[0] = lax.dot_general(fs, wv_ref[...], cdims,
                               preferred_element_type=_F32) + bv_ref[...]
    pqr_ref[0] = lax.dot_general(pq_ref[0], eye_ref[...], cdims,
                                 preferred_element_type=_F32)
    psr_ref[0] = lax.dot_general(ps_ref[0], eye_ref[...], cdims,
                                 preferred_element_type=_F32)


def _proj(fq, fs, pq, ps, wq, wk, wv, bq, bk, bv):
    b, c, n = fq.shape
    m = fs.shape[2]
    eye = jnp.eye(16, 3, dtype=_F32)
    full = lambda s: pl.BlockSpec(s, lambda i: (0,) * len(s))
    return pl.pallas_call(
        _proj_body,
        grid=(b,),
        in_specs=[
            pl.BlockSpec((1, c, n), lambda i: (i, 0, 0)),
            pl.BlockSpec((1, c, m), lambda i: (i, 0, 0)),
            pl.BlockSpec((1, 3, n), lambda i: (i, 0, 0)),
            pl.BlockSpec((1, 3, m), lambda i: (i, 0, 0)),
            full((64, c)), full((64, c)), full((64, c)),
            full((1, 64)), full((1, 64)), full((1, 64)),
            full((16, 3)),
        ],
        out_specs=[
            pl.BlockSpec((1, n, 64), lambda i: (i, 0, 0)),
            pl.BlockSpec((1, m, 64), lambda i: (i, 0, 0)),
            pl.BlockSpec((1, m, 64), lambda i: (i, 0, 0)),
            pl.BlockSpec((1, n, 16), lambda i: (i, 0, 0)),
            pl.BlockSpec((1, m, 16), lambda i: (i, 0, 0)),
        ],
        out_shape=[
            jax.ShapeDtypeStruct((b, n, 64), _F32),
            jax.ShapeDtypeStruct((b, m, 64), _F32),
            jax.ShapeDtypeStruct((b, m, 64), _F32),
            jax.ShapeDtypeStruct((b, n, 16), _F32),
            jax.ShapeDtypeStruct((b, m, 16), _F32),
        ],
    )(fq, fs, pq, ps, wq, wk, wv, bq[None], bk[None], bv[None], eye)


def _knn_body(k, m, pq_ref, ps_ref, idx_ref):
    tn = pq_ref.shape[2]
    pq = pq_ref[0]
    ps = ps_ref[0]
    ones = jnp.ones((1, 3), _F32)
    cross = lax.dot_general(pq, ps, (((0,), (0,)), ((), ())),
                            preferred_element_type=_F32)
    qn = lax.dot_general(pq * pq, ones, (((0,), (1,)), ((), ())),
                         preferred_element_type=_F32)
    sn = jnp.sum(ps * ps, axis=0, keepdims=True)
    d = jnp.maximum(qn + (sn - 2.0 * cross), 0.0)
    iota = lax.broadcasted_iota(jnp.int32, (tn, m), 1)
    key = jnp.bitwise_or(
        jnp.bitwise_and(lax.bitcast_convert_type(d, jnp.int32), ~0xFFF),
        iota)
    cols = []
    for j in range(k):
        kmin = jnp.min(key, axis=1, keepdims=True)
        cols.append(kmin)
        if j + 1 < k:
            key = jnp.where(key == kmin, jnp.int32(0x7FFFFFFF), key)
    bi = pl.program_id(0)
    idx_ref[0] = jnp.bitwise_and(jnp.concatenate(cols, axis=1), 0xFFF) + bi * m


def _knn(k, pq, ps):
    b, _, n = pq.shape
    m = ps.shape[2]
    tn = min(_KNN_TN, n)
    return pl.pallas_call(
        functools.partial(_knn_body, k, m),
        grid=(b, n // tn),
        in_specs=[
            pl.BlockSpec((1, 3, tn), lambda i, j: (i, 0, j)),
            pl.BlockSpec((1, 3, m), lambda i, j: (i, 0, 0)),
        ],
        out_specs=pl.BlockSpec((1, tn, k), lambda i, j: (i, j, 0)),
        out_shape=jax.ShapeDtypeStruct((b, n, k), jnp.int32),
    )(pq, ps)


def _sc_gather(ke_t, v_t, ps_t, idx_flat):
    r = idx_flat.shape[0]
    nw = 32
    ch = 128
    n_per_w = r // nw
    n_it = n_per_w // ch
    mesh = plsc.VectorSubcoreMesh(core_axis_name="c", subcore_axis_name="s")

    @functools.partial(
        pl.kernel, mesh=mesh,
        compiler_params=pltpu.CompilerParams(use_tc_tiling_on_sc=False),
        out_type=(jax.ShapeDtypeStruct((r, 64), _F32),
                  jax.ShapeDtypeStruct((r, 64), _F32),
                  jax.ShapeDtypeStruct((r, 16), _F32)),
        scratch_types=[
            pltpu.VMEM((ch,), jnp.int32),
            pltpu.VMEM((ch, 64), _F32),
            pltpu.VMEM((ch, 64), _F32),
            pltpu.VMEM((ch, 16), _F32),
            pltpu.SemaphoreType.DMA,
            pltpu.SemaphoreType.DMA,
            pltpu.SemaphoreType.DMA,
        ],
    )
    def gk(ke_hbm, v_hbm, ps_hbm, idx_hbm, oke, ov, ops,
           idx_v, ke_v, v_v, ps_v, s1, s2, s3):
        wid = lax.axis_index("s") * 2 + lax.axis_index("c")
        base = wid * n_per_w

        def body(i, carry):
            off = pl.multiple_of(base + i * ch, ch)
            pltpu.sync_copy(idx_hbm.at[pl.ds(off, ch)], idx_v)
            c1 = pltpu.async_copy(ke_hbm.at[idx_v], ke_v, s1)
            c2 = pltpu.async_copy(v_hbm.at[idx_v], v_v, s2)
            c3 = pltpu.async_copy(ps_hbm.at[idx_v], ps_v, s3)
            c1.wait()
            c2.wait()
            c3.wait()
            pltpu.sync_copy(ke_v, oke.at[pl.ds(off, ch)])
            pltpu.sync_copy(v_v, ov.at[pl.ds(off, ch)])
            pltpu.sync_copy(ps_v, ops.at[pl.ds(off, ch)])
            return carry

        lax.fori_loop(0, n_it, body, 0)

    return gk(ke_t, v_t, ps_t, idx_flat)


def _mlp_body(k, q_ref, pqr_ref, kg_ref, vg_ref, psg_ref, fq_ref,
              pw1_ref, s1_ref, t1_ref, pw2_ref, pb2_ref,
              aw1_ref, s2_ref, t2_ref, aw2_ref, ab2_ref,
              ew_ref, eb_ref, out_ref):
    tn = q_ref.shape[1]
    ktn = k * tn
    cT = (((1,), (1,)), ((), ()))
    kg = kg_ref[0].reshape(ktn, 64)
    psg = psg_ref[0].reshape(ktn, 16)
    qrep = jnp.broadcast_to(q_ref[0][None], (k, tn, 64)).reshape(ktn, 64)
    pqrep = jnp.broadcast_to(pqr_ref[0][None], (k, tn, 16)).reshape(ktn, 16)

    psrel = pqrep - psg
    pe = lax.dot_general(psrel, pw1_ref[...], cT,
                         preferred_element_type=_F32)
    pe = jnp.maximum(pe * s1_ref[...] + t1_ref[...], 0.0)
    pe = lax.dot_general(pe, pw2_ref[...], cT,
                         preferred_element_type=_F32) + pb2_ref[...]

    ain = qrep - kg + pe
    h = lax.dot_general(ain, aw1_ref[...], cT, preferred_element_type=_F32)
    h = jnp.maximum(h * s2_ref[...] + t2_ref[...], 0.0)
    a = lax.dot_general(h, aw2_ref[...], cT,
                        preferred_element_type=_F32) + ab2_ref[...]

    a3 = a.reshape(k, tn, 64)
    mx = jnp.max(a3, axis=0)
    e3 = jnp.exp(a3 - mx[None])
    w3 = e3 / jnp.sum(e3, axis=0)[None]
    vpe = vg_ref[0] + pe.reshape(k, tn, 64)
    agg = jnp.sum(w3 * vpe, axis=0)

    outc = lax.dot_general(ew_ref[...], agg, cT,
                           preferred_element_type=_F32)
    out_ref[0] = outc + eb_ref[...] + fq_ref[0]


def _mlp(k, q, pqr, gke, gv, gps, fq, pw1, s1, t1, pw2, pb2,
         aw1, s2, t2, aw2, ab2, ew, eb):
    b, c, n = fq.shape
    tn = min(_MLP_TN, n)
    full = lambda s: pl.BlockSpec(s, lambda i, j: (0,) * len(s))
    return pl.pallas_call(
        functools.partial(_mlp_body, k),
        grid=(b, n // tn),
        in_specs=[
            pl.BlockSpec((1, tn, 64), lambda i, j: (i, j, 0)),
            pl.BlockSpec((1, tn, 16), lambda i, j: (i, j, 0)),
            pl.BlockSpec((1, k, tn, 64), lambda i, j: (i, 0, j, 0)),
            pl.BlockSpec((1, k, tn, 64), lambda i, j: (i, 0, j, 0)),
            pl.BlockSpec((1, k, tn, 16), lambda i, j: (i, 0, j, 0)),
            pl.BlockSpec((1, c, tn), lambda i, j: (i, 0, j)),
            full((64, 16)), full((1, 64)), full((1, 64)),
            full((64, 64)), full((1, 64)),
            full((256, 64)), full((1, 256)), full((1, 256)),
            full((64, 256)), full((1, 64)),
            full((c, 64)), full((c, 1)),
        ],
        out_specs=pl.BlockSpec((1, c, tn), lambda i, j: (i, 0, j)),
        out_shape=jax.ShapeDtypeStruct((b, c, n), _F32),
    )(q, pqr, gke, gv, gps, fq,
      pw1, s1, t1, pw2, pb2, aw1, s2, t2, aw2, ab2, ew, eb)


def _fold_bn(g, be, m, v, bias):
    scale = g * lax.rsqrt(v + 1e-5)
    shift = be - m * scale + bias * scale
    return scale[None], shift[None]


def _vfa_scale(p, k, pq, fq, ps, fs):
    b, c, n = fq.shape
    m = fs.shape[2]

    q, ke, v, pqr, psr = _proj(fq, fs, pq, ps,
                               p['wq'], p['wk'], p['wv'],
                               p['bq'], p['bk'], p['bv'])
    idx = _knn(k, pq, ps)
    idx_flat = jnp.transpose(idx, (0, 2, 1)).reshape(b * k * n)
    gke, gv, gps = _sc_gather(ke.reshape(b * m, 64),
                              v.reshape(b * m, 64),
                              psr.reshape(b * m, 16), idx_flat)
    gke = gke.reshape(b, k, n, 64)
    gv = gv.reshape(b, k, n, 64)
    gps = gps.reshape(b, k, n, 16)

    pw1 = jnp.pad(p['pw1'], ((0, 0), (0, 13)))
    s1, t1 = _fold_bn(p['pg'], p['pbe'], p['pm'], p['pv'], p['pb1'])
    s2, t2 = _fold_bn(p['ag'], p['abe'], p['am'], p['av'], p['ab1'])
    return _mlp(k, q, pqr, gke, gv, gps, fq,
                pw1, s1, t1, p['pw2'], p['pb2'][None],
                p['aw1'], s2, t2, p['aw2'], p['ab2'][None],
                p['ew'], p['eb'][:, None])


def kernel(pp0, pp1, pp2, pf0, pf1, pf2, gp0, gp1, gp2, gf0, gf1, gf2, params):
    pp = [pp0, pp1, pp2]
    pf = [pf0, pf1, pf2]
    gp = [gp0, gp1, gp2]
    gf = [gf0, gf1, gf2]
    knns = [16, 12, 8]
    pre_pos = []
    pre_f = []
    for i in range(2, -1, -1):
        f = _vfa_scale(params[i], knns[i], pp[i], pf[i], gp[i], gf[i])
        pre_pos.append(pp[i])
        pre_f.append(f)
    return (tuple(pre_pos), tuple(pre_f))


# speedup vs baseline: 19.3565x; 1.0020x over previous
import functools

import jax
import jax.numpy as jnp
from jax import lax
from jax.experimental import pallas as pl
from jax.experimental.pallas import tpu as pltpu
from jax.experimental.pallas import tpu_sc as plsc

_F32 = jnp.float32
_KNN_TN = 128
_MLP_TN = 128


def _proj_body(fq_ref, fs_ref, pq_ref, ps_ref, wq_ref, wk_ref, wv_ref,
               bq_ref, bk_ref, bv_ref, eye_ref,
               q_ref, ke_ref, v_ref, pqr_ref, psr_ref):
    cdims = (((0,), (1,)), ((), ()))
    fq = fq_ref[0]
    fs = fs_ref[0]
    q_ref[0] = lax.dot_general(fq, wq_ref[...], cdims,
                               preferred_element_type=_F32) + bq_ref[...]
    ke_ref[0] = lax.dot_general(fs, wk_ref[...], cdims,
                                preferred_element_type=_F32) + bk_ref[...]
    v_ref[0] = lax.dot_general(fs, wv_ref[...], cdims,
                               preferred_element_type=_F32) + bv_ref[...]
    pqr_ref[0] = lax.dot_general(pq_ref[0], eye_ref[...], cdims,
                                 preferred_element_type=_F32)
    psr_ref[0] = lax.dot_general(ps_ref[0], eye_ref[...], cdims,
                                 preferred_element_type=_F32)


def _proj(fq, fs, pq, ps, wq, wk, wv, bq, bk, bv):
    b, c, n = fq.shape
    m = fs.shape[2]
    eye = jnp.eye(16, 3, dtype=_F32)
    full = lambda s: pl.BlockSpec(s, lambda i: (0,) * len(s))
    return pl.pallas_call(
        _proj_body,
        grid=(b,),
        in_specs=[
            pl.BlockSpec((1, c, n), lambda i: (i, 0, 0)),
            pl.BlockSpec((1, c, m), lambda i: (i, 0, 0)),
            pl.BlockSpec((1, 3, n), lambda i: (i, 0, 0)),
            pl.BlockSpec((1, 3, m), lambda i: (i, 0, 0)),
            full((64, c)), full((64, c)), full((64, c)),
            full((1, 64)), full((1, 64)), full((1, 64)),
            full((16, 3)),
        ],
        out_specs=[
            pl.BlockSpec((1, n, 64), lambda i: (i, 0, 0)),
            pl.BlockSpec((1, m, 64), lambda i: (i, 0, 0)),
            pl.BlockSpec((1, m, 64), lambda i: (i, 0, 0)),
            pl.BlockSpec((1, n, 16), lambda i: (i, 0, 0)),
            pl.BlockSpec((1, m, 16), lambda i: (i, 0, 0)),
        ],
        out_shape=[
            jax.ShapeDtypeStruct((b, n, 64), _F32),
            jax.ShapeDtypeStruct((b, m, 64), _F32),
            jax.ShapeDtypeStruct((b, m, 64), _F32),
            jax.ShapeDtypeStruct((b, n, 16), _F32),
            jax.ShapeDtypeStruct((b, m, 16), _F32),
        ],
    )(fq, fs, pq, ps, wq, wk, wv, bq[None], bk[None], bv[None], eye)


def _knn_body(k, m, pq_ref, ps_ref, idx_ref):
    tn = pq_ref.shape[2]
    pq = pq_ref[0]
    ps = ps_ref[0]
    ones = jnp.ones((1, 3), _F32)
    cross = lax.dot_general(pq, ps, (((0,), (0,)), ((), ())),
                            preferred_element_type=_F32)
    qn = lax.dot_general(pq * pq, ones, (((0,), (1,)), ((), ())),
                         preferred_element_type=_F32)
    sn = jnp.sum(ps * ps, axis=0, keepdims=True)
    d = jnp.maximum(qn + (sn - 2.0 * cross), 0.0)
    iota = lax.broadcasted_iota(jnp.int32, (tn, m), 1)
    key = jnp.bitwise_or(
        jnp.bitwise_and(lax.bitcast_convert_type(d, jnp.int32), ~0xFFF),
        iota)
    cols = []
    for j in range(k):
        kmin = jnp.min(key, axis=1, keepdims=True)
        cols.append(kmin)
        if j + 1 < k:
            key = jnp.where(key == kmin, jnp.int32(0x7FFFFFFF), key)
    bi = pl.program_id(0)
    idx_ref[0] = jnp.bitwise_and(jnp.concatenate(cols, axis=1), 0xFFF) + bi * m


def _knn(k, pq, ps):
    b, _, n = pq.shape
    m = ps.shape[2]
    tn = min(_KNN_TN, n)
    return pl.pallas_call(
        functools.partial(_knn_body, k, m),
        grid=(b, n // tn),
        in_specs=[
            pl.BlockSpec((1, 3, tn), lambda i, j: (i, 0, j)),
            pl.BlockSpec((1, 3, m), lambda i, j: (i, 0, 0)),
        ],
        out_specs=pl.BlockSpec((1, tn, k), lambda i, j: (i, j, 0)),
        out_shape=jax.ShapeDtypeStruct((b, n, k), jnp.int32),
    )(pq, ps)


def _sc_gather(ke_t, v_t, ps_t, idx_flat):
    r = idx_flat.shape[0]
    nw = 32
    ch = 128
    n_per_w = r // nw
    n_it = n_per_w // ch
    mesh = plsc.VectorSubcoreMesh(core_axis_name="c", subcore_axis_name="s")

    @functools.partial(
        pl.kernel, mesh=mesh,
        compiler_params=pltpu.CompilerParams(use_tc_tiling_on_sc=False),
        out_type=(jax.ShapeDtypeStruct((r, 64), _F32),
                  jax.ShapeDtypeStruct((r, 64), _F32),
                  jax.ShapeDtypeStruct((r, 16), _F32)),
        scratch_types=[
            pltpu.VMEM((ch,), jnp.int32),
            pltpu.VMEM((ch, 64), _F32),
            pltpu.VMEM((ch, 64), _F32),
            pltpu.VMEM((ch, 16), _F32),
            pltpu.SemaphoreType.DMA,
            pltpu.SemaphoreType.DMA,
            pltpu.SemaphoreType.DMA,
        ],
    )
    def gk(ke_hbm, v_hbm, ps_hbm, idx_hbm, oke, ov, ops,
           idx_v, ke_v, v_v, ps_v, s1, s2, s3):
        wid = lax.axis_index("s") * 2 + lax.axis_index("c")
        base = wid * n_per_w

        def body(i, carry):
            off = pl.multiple_of(base + i * ch, ch)
            pltpu.sync_copy(idx_hbm.at[pl.ds(off, ch)], idx_v)
            c1 = pltpu.async_copy(ke_hbm.at[idx_v], ke_v, s1)
            c2 = pltpu.async_copy(v_hbm.at[idx_v], v_v, s2)
            c3 = pltpu.async_copy(ps_hbm.at[idx_v], ps_v, s3)
            c1.wait()
            c2.wait()
            c3.wait()
            pltpu.sync_copy(ke_v, oke.at[pl.ds(off, ch)])
            pltpu.sync_copy(v_v, ov.at[pl.ds(off, ch)])
            pltpu.sync_copy(ps_v, ops.at[pl.ds(off, ch)])
            return carry

        lax.fori_loop(0, n_it, body, 0)

    return gk(ke_t, v_t, ps_t, idx_flat)


def _mlp_body(k, q_ref, pqr_ref, kg_ref, vg_ref, psg_ref, fq_ref,
              pw1_ref, s1_ref, t1_ref, pw2_ref, pb2_ref,
              aw1_ref, s2_ref, t2_ref, aw2_ref, ab2_ref,
              ew_ref, eb_ref, out_ref):
    tn = q_ref.shape[1]
    ktn = k * tn
    cT = (((1,), (1,)), ((), ()))
    bf = jnp.bfloat16
    kg = kg_ref[0].reshape(ktn, 64)
    psg = psg_ref[0].reshape(ktn, 16)
    qrep = jnp.broadcast_to(q_ref[0][None], (k, tn, 64)).reshape(ktn, 64)
    pqrep = jnp.broadcast_to(pqr_ref[0][None], (k, tn, 16)).reshape(ktn, 16)

    psrel = pqrep - psg
    pe = lax.dot_general(psrel.astype(bf), pw1_ref[...].astype(bf), cT,
                         preferred_element_type=_F32)
    pe = jnp.maximum(pe * s1_ref[...] + t1_ref[...], 0.0)
    pe = lax.dot_general(pe.astype(bf), pw2_ref[...].astype(bf), cT,
                         preferred_element_type=_F32) + pb2_ref[...]

    ain = qrep - kg + pe
    h = lax.dot_general(ain.astype(bf), aw1_ref[...].astype(bf), cT,
                        preferred_element_type=_F32)
    h = jnp.maximum(h * s2_ref[...] + t2_ref[...], 0.0)
    a = lax.dot_general(h.astype(bf), aw2_ref[...].astype(bf), cT,
                        preferred_element_type=_F32) + ab2_ref[...]

    a3 = a.reshape(k, tn, 64)
    mx = jnp.max(a3, axis=0)
    e3 = jnp.exp(a3 - mx[None])
    w3 = e3 / jnp.sum(e3, axis=0)[None]
    vpe = vg_ref[0] + pe.reshape(k, tn, 64)
    agg = jnp.sum(w3 * vpe, axis=0)

    outc = lax.dot_general(ew_ref[...].astype(bf), agg.astype(bf), cT,
                           preferred_element_type=_F32)
    out_ref[0] = outc + eb_ref[...] + fq_ref[0]


def _mlp(k, q, pqr, gke, gv, gps, fq, pw1, s1, t1, pw2, pb2,
         aw1, s2, t2, aw2, ab2, ew, eb):
    b, c, n = fq.shape
    tn = min(_MLP_TN, n)
    full = lambda s: pl.BlockSpec(s, lambda i, j: (0,) * len(s))
    return pl.pallas_call(
        functools.partial(_mlp_body, k),
        grid=(b, n // tn),
        in_specs=[
            pl.BlockSpec((1, tn, 64), lambda i, j: (i, j, 0)),
            pl.BlockSpec((1, tn, 16), lambda i, j: (i, j, 0)),
            pl.BlockSpec((1, k, tn, 64), lambda i, j: (i, 0, j, 0)),
            pl.BlockSpec((1, k, tn, 64), lambda i, j: (i, 0, j, 0)),
            pl.BlockSpec((1, k, tn, 16), lambda i, j: (i, 0, j, 0)),
            pl.BlockSpec((1, c, tn), lambda i, j: (i, 0, j)),
            full((64, 16)), full((1, 64)), full((1, 64)),
            full((64, 64)), full((1, 64)),
            full((256, 64)), full((1, 256)), full((1, 256)),
            full((64, 256)), full((1, 64)),
            full((c, 64)), full((c, 1)),
        ],
        out_specs=pl.BlockSpec((1, c, tn), lambda i, j: (i, 0, j)),
        out_shape=jax.ShapeDtypeStruct((b, c, n), _F32),
    )(q, pqr, gke, gv, gps, fq,
      pw1, s1, t1, pw2, pb2, aw1, s2, t2, aw2, ab2, ew, eb)


def _fold_bn(g, be, m, v, bias):
    scale = g * lax.rsqrt(v + 1e-5)
    shift = be - m * scale + bias * scale
    return scale[None], shift[None]


def _vfa_scale(p, k, pq, fq, ps, fs):
    b, c, n = fq.shape
    m = fs.shape[2]

    q, ke, v, pqr, psr = _proj(fq, fs, pq, ps,
                               p['wq'], p['wk'], p['wv'],
                               p['bq'], p['bk'], p['bv'])
    idx = _knn(k, pq, ps)
    idx_flat = jnp.transpose(idx, (0, 2, 1)).reshape(b * k * n)
    gke, gv, gps = _sc_gather(ke.reshape(b * m, 64),
                              v.reshape(b * m, 64),
                              psr.reshape(b * m, 16), idx_flat)
    gke = gke.reshape(b, k, n, 64)
    gv = gv.reshape(b, k, n, 64)
    gps = gps.reshape(b, k, n, 16)

    pw1 = jnp.pad(p['pw1'], ((0, 0), (0, 13)))
    s1, t1 = _fold_bn(p['pg'], p['pbe'], p['pm'], p['pv'], p['pb1'])
    s2, t2 = _fold_bn(p['ag'], p['abe'], p['am'], p['av'], p['ab1'])
    return _mlp(k, q, pqr, gke, gv, gps, fq,
                pw1, s1, t1, p['pw2'], p['pb2'][None],
                p['aw1'], s2, t2, p['aw2'], p['ab2'][None],
                p['ew'], p['eb'][:, None])


def kernel(pp0, pp1, pp2, pf0, pf1, pf2, gp0, gp1, gp2, gf0, gf1, gf2, params):
    pp = [pp0, pp1, pp2]
    pf = [pf0, pf1, pf2]
    gp = [gp0, gp1, gp2]
    gf = [gf0, gf1, gf2]
    knns = [16, 12, 8]
    pre_pos = []
    pre_f = []
    for i in range(2, -1, -1):
        f = _vfa_scale(params[i], knns[i], pp[i], pf[i], gp[i], gf[i])
        pre_pos.append(pp[i])
        pre_f.append(f)
    return (tuple(pre_pos), tuple(pre_f))
